# Initial kernel scaffold; baseline (speedup 1.0000x reference)
import jax
import jax.numpy as jnp
from jax.experimental import pallas as pl


def kernel(z):
    raise NotImplementedError("write your pallas kernel here")


# speedup vs baseline: 21.2761x; 21.2761x over previous
import functools

import jax
import jax.numpy as jnp
from jax.experimental import pallas as pl


_MICHELOT_ITERS = 12


def _entmax_block(z_ref, out_ref):
    z = z_ref[...]
    rows, n = z.shape
    zmax = jnp.max(z, axis=1, keepdims=True)
    t0 = zmax - 1.0

    def step(_, t):
        m = z > t
        c = jnp.sum(m.astype(jnp.float32), axis=1, keepdims=True)
        s = jnp.sum(jnp.where(m, z, 0.0), axis=1, keepdims=True)
        return (s - 1.0) / c

    t = jax.lax.fori_loop(0, _MICHELOT_ITERS, step, t0)

    m = z > t
    ci = jnp.sum(m.astype(jnp.int32), axis=1, keepdims=True)
    cf = ci.astype(jnp.float32)
    s = jnp.sum(jnp.where(m, z, 0.0), axis=1, keepdims=True)
    z_next = jnp.max(jnp.where(m, -jnp.inf, z), axis=1, keepdims=True)
    gathered = s + jnp.where(ci < n, z_next, 0.0)
    tau = (gathered - 1.0) / cf
    col = jax.lax.broadcasted_iota(jnp.int32, (rows, n), 1)
    tau_full = jnp.where(col < ci, tau, 0.0)
    r = jnp.maximum(z - tau_full, 0.0)
    out_ref[...] = r * jnp.sqrt(r)


@jax.jit
def kernel(z):
    b, n = z.shape
    rows = 8
    return pl.pallas_call(
        _entmax_block,
        out_shape=jax.ShapeDtypeStruct((b, n), z.dtype),
        grid=(b // rows,),
        in_specs=[pl.BlockSpec((rows, n), lambda i: (i, 0))],
        out_specs=pl.BlockSpec((rows, n), lambda i: (i, 0)),
    )(z)


# speedup vs baseline: 26.7115x; 1.2555x over previous
import jax
import jax.numpy as jnp
from jax.experimental import pallas as pl
from jax.experimental.pallas import tpu as pltpu


_MICHELOT_ITERS = 9


def _entmax_block(z_ref, out_ref):
    z = z_ref[...]
    rows, n = z.shape
    zmax = jnp.max(z, axis=1, keepdims=True)
    t0 = zmax - 1.0
    zeros = jnp.zeros_like(t0)

    def step(_, carry):
        t, _, _ = carry
        m = z > t
        c = jnp.sum(jnp.where(m, 1.0, 0.0), axis=1, keepdims=True)
        s = jnp.sum(jnp.where(m, z, 0.0), axis=1, keepdims=True)
        return (s - 1.0) / c, c, s

    t, cf, s = jax.lax.fori_loop(0, _MICHELOT_ITERS, step, (t0, zeros, zeros))

    z_next = jnp.max(jnp.where(z > t, -jnp.inf, z), axis=1, keepdims=True)
    ci = cf.astype(jnp.int32)
    gathered = s + jnp.where(ci < n, z_next, 0.0)
    tau = (gathered - 1.0) / cf
    col = jax.lax.broadcasted_iota(jnp.int32, (rows, n), 1)
    tau_full = jnp.where(col < ci, tau, 0.0)
    r = jnp.maximum(z - tau_full, 0.0)
    out_ref[...] = r * jnp.sqrt(r)


@jax.jit
def kernel(z):
    b, n = z.shape
    rows = 8
    return pl.pallas_call(
        _entmax_block,
        out_shape=jax.ShapeDtypeStruct((b, n), z.dtype),
        grid=(b // rows,),
        in_specs=[pl.BlockSpec((rows, n), lambda i: (i, 0))],
        out_specs=pl.BlockSpec((rows, n), lambda i: (i, 0)),
        compiler_params=pltpu.CompilerParams(
            dimension_semantics=("parallel",),
        ),
    )(z)


# speedup vs baseline: 31.7893x; 1.1901x over previous
import jax
import jax.numpy as jnp
from jax.experimental import pallas as pl
from jax.experimental.pallas import tpu as pltpu


_MAX_ITERS = 16


def _entmax_block(z_ref, out_ref):
    z = z_ref[...]
    rows, n = z.shape
    zmax = jnp.max(z, axis=1, keepdims=True)
    t0 = zmax - 1.0
    zeros = jnp.zeros_like(t0)

    def cond(carry):
        i, t, t_prev, _, _ = carry
        return jnp.logical_and(i < _MAX_ITERS, jnp.any(t != t_prev))

    def step(carry):
        i, t, _, _, _ = carry
        m = z > t
        c = jnp.sum(jnp.where(m, 1.0, 0.0), axis=1, keepdims=True)
        s = jnp.sum(jnp.where(m, z, 0.0), axis=1, keepdims=True)
        return i + 1, (s - 1.0) / c, t, c, s

    _, t, _, cf, s = jax.lax.while_loop(
        cond, step, (0, t0, t0 - 1.0, zeros, zeros))

    z_next = jnp.max(jnp.where(z > t, -jnp.inf, z), axis=1, keepdims=True)
    ci = cf.astype(jnp.int32)
    gathered = s + jnp.where(ci < n, z_next, 0.0)
    tau = (gathered - 1.0) / cf
    col = jax.lax.broadcasted_iota(jnp.int32, (rows, n), 1)
    tau_full = jnp.where(col < ci, tau, 0.0)
    r = jnp.maximum(z - tau_full, 0.0)
    out_ref[...] = (r * r) * jax.lax.rsqrt(jnp.maximum(r, 1e-30))


@jax.jit
def kernel(z):
    b, n = z.shape
    rows = 8
    return pl.pallas_call(
        _entmax_block,
        out_shape=jax.ShapeDtypeStruct((b, n), z.dtype),
        grid=(b // rows,),
        in_specs=[pl.BlockSpec((rows, n), lambda i: (i, 0))],
        out_specs=pl.BlockSpec((rows, n), lambda i: (i, 0)),
        compiler_params=pltpu.CompilerParams(
            dimension_semantics=("parallel",),
        ),
    )(z)


# speedup vs baseline: 51.3896x; 1.6166x over previous
import jax
import jax.numpy as jnp
from jax.experimental import pallas as pl
from jax.experimental.pallas import tpu as pltpu


_MAX_ITERS = 16


def _entmax_block(z_ref, out_ref):
    z = z_ref[...]
    rows, n = z.shape
    m1 = jnp.max(z.reshape(rows, n // 128, 128), axis=1)
    zmax = jnp.max(m1, axis=1, keepdims=True)
    t0 = zmax - 1.0
    zeros = jnp.zeros_like(t0)

    def warm_step(_, tt):
        mm = m1 > tt
        cc = jnp.sum(jnp.where(mm, 1.0, 0.0), axis=1, keepdims=True)
        ss = jnp.sum(jnp.where(mm, m1, 0.0), axis=1, keepdims=True)
        return (ss - 1.0) / cc

    t0 = jax.lax.fori_loop(0, 8, warm_step, t0)

    def cond(carry):
        i, t, t_prev, _, _ = carry
        return jnp.logical_and(i < _MAX_ITERS, jnp.any(t != t_prev))

    def step(carry):
        i, t, _, _, _ = carry
        m = z > t
        c = jnp.sum(jnp.where(m, 1.0, 0.0), axis=1, keepdims=True)
        s = jnp.sum(jnp.where(m, z, 0.0), axis=1, keepdims=True)
        return i + 1, (s - 1.0) / c, t, c, s

    _, t, _, cf, s = jax.lax.while_loop(
        cond, step, (0, t0, t0 - 1.0, zeros, zeros))

    z_next = jnp.max(jnp.where(z > t, -jnp.inf, z), axis=1, keepdims=True)
    ci = cf.astype(jnp.int32)
    gathered = s + jnp.where(ci < n, z_next, 0.0)
    tau = (gathered - 1.0) / cf
    col = jax.lax.broadcasted_iota(jnp.int32, (rows, n), 1)
    tau_full = jnp.where(col < ci, tau, 0.0)
    r = jnp.maximum(z - tau_full, 0.0)
    out_ref[...] = (r * r) * jax.lax.rsqrt(jnp.maximum(r, 1e-30))


@jax.jit
def kernel(z):
    b, n = z.shape
    rows = 8
    return pl.pallas_call(
        _entmax_block,
        out_shape=jax.ShapeDtypeStruct((b, n), z.dtype),
        grid=(b // rows,),
        in_specs=[pl.BlockSpec((rows, n), lambda i: (i, 0))],
        out_specs=pl.BlockSpec((rows, n), lambda i: (i, 0)),
        compiler_params=pltpu.CompilerParams(
            dimension_semantics=("parallel",),
        ),
    )(z)


# speedup vs baseline: 68.0055x; 1.3233x over previous
import jax
import jax.numpy as jnp
from jax.experimental import pallas as pl
from jax.experimental.pallas import tpu as pltpu


_MAX_ITERS = 16


def _entmax_block(z_ref, out_ref):
    z = z_ref[...]
    rows, n = z.shape
    m1 = jnp.max(z.reshape(rows, n // 128, 128), axis=1)
    zmax = jnp.max(m1, axis=1, keepdims=True)
    t0 = zmax - 1.0
    zeros = jnp.zeros_like(t0)

    def warm_step(_, tt):
        mm = m1 > tt
        cc = jnp.sum(jnp.where(mm, 1.0, 0.0), axis=1, keepdims=True)
        ss = jnp.sum(jnp.where(mm, m1, 0.0), axis=1, keepdims=True)
        return (ss - 1.0) / cc

    t0 = jax.lax.fori_loop(0, 8, warm_step, t0)

    def cond(carry):
        i, t, t_prev, _, _ = carry
        return jnp.logical_and(i < _MAX_ITERS, jnp.any(t != t_prev))

    def step(carry):
        i, t, _, _, _ = carry
        m = z > t
        c = jnp.sum(jnp.where(m, 1.0, 0.0), axis=1, keepdims=True)
        s = jnp.sum(jnp.where(m, z, 0.0), axis=1, keepdims=True)
        return i + 1, (s - 1.0) / c, t, c, s

    _, t, _, cf, s = jax.lax.while_loop(
        cond, step, (0, t0, t0 - 1.0, zeros, zeros))

    z_next = jnp.max(jnp.where(z > t, -jnp.inf, z), axis=1, keepdims=True)
    ci = cf.astype(jnp.int32)
    gathered = s + jnp.where(ci < n, z_next, 0.0)
    tau = (gathered - 1.0) / cf
    col = jax.lax.broadcasted_iota(jnp.int32, (rows, n), 1)
    tau_full = jnp.where(col < ci, tau, 0.0)
    r = jnp.maximum(z - tau_full, 0.0)
    out_ref[...] = (r * r) * jax.lax.rsqrt(jnp.maximum(r, 1e-30))


@jax.jit
def kernel(z):
    b, n = z.shape
    rows = 16
    return pl.pallas_call(
        _entmax_block,
        out_shape=jax.ShapeDtypeStruct((b, n), z.dtype),
        grid=(b // rows,),
        in_specs=[pl.BlockSpec((rows, n), lambda i: (i, 0))],
        out_specs=pl.BlockSpec((rows, n), lambda i: (i, 0)),
        compiler_params=pltpu.CompilerParams(
            dimension_semantics=("parallel",),
        ),
    )(z)


# speedup vs baseline: 71.7075x; 1.0544x over previous
import jax
import jax.numpy as jnp
from jax.experimental import pallas as pl
from jax.experimental.pallas import tpu as pltpu


_MAX_ITERS = 16


def _entmax_block(z_ref, out_ref):
    z = z_ref[...]
    rows, n = z.shape
    m1 = jnp.max(z.reshape(rows, n // 128, 128), axis=1)
    zmax = jnp.max(m1, axis=1, keepdims=True)
    t0 = zmax - 1.0
    zeros = jnp.zeros_like(t0)

    def warm_step(_, tt):
        mm = m1 > tt
        cc = jnp.sum(jnp.where(mm, 1.0, 0.0), axis=1, keepdims=True)
        ss = jnp.sum(jnp.where(mm, m1, 0.0), axis=1, keepdims=True)
        return (ss - 1.0) / cc

    t0 = jax.lax.fori_loop(0, 8, warm_step, t0)

    def cond(carry):
        i, t, t_prev, _, _ = carry
        return jnp.logical_and(i < _MAX_ITERS, jnp.any(t != t_prev))

    def step(carry):
        i, t, _, _, _ = carry
        m = z > t
        c = jnp.sum(jnp.where(m, 1.0, 0.0), axis=1, keepdims=True)
        s = jnp.sum(jnp.where(m, z, 0.0), axis=1, keepdims=True)
        return i + 1, (s - 1.0) / c, t, c, s

    _, t, _, cf, s = jax.lax.while_loop(
        cond, step, (0, t0, t0 - 1.0, zeros, zeros))

    z_next = jnp.max(jnp.where(z > t, -jnp.inf, z), axis=1, keepdims=True)
    ci = cf.astype(jnp.int32)
    gathered = s + jnp.where(ci < n, z_next, 0.0)
    tau = (gathered - 1.0) / cf
    col = jax.lax.broadcasted_iota(jnp.int32, (rows, n), 1)
    tau_full = jnp.where(col < ci, tau, 0.0)
    r = jnp.maximum(z - tau_full, 0.0)
    out_ref[...] = (r * r) * jax.lax.rsqrt(jnp.maximum(r, 1e-30))


@jax.jit
def kernel(z):
    b, n = z.shape
    rows = 32
    return pl.pallas_call(
        _entmax_block,
        out_shape=jax.ShapeDtypeStruct((b, n), z.dtype),
        grid=(b // rows,),
        in_specs=[pl.BlockSpec((rows, n), lambda i: (i, 0))],
        out_specs=pl.BlockSpec((rows, n), lambda i: (i, 0)),
        compiler_params=pltpu.CompilerParams(
            dimension_semantics=("parallel",),
        ),
    )(z)
